# Initial kernel scaffold; baseline (speedup 1.0000x reference)
import jax
import jax.numpy as jnp
from jax.experimental import pallas as pl


def kernel(x, gi, ee):
    raise NotImplementedError("write your pallas kernel here")


# speedup vs baseline: 3.5941x; 3.5941x over previous
import functools

import jax
import jax.numpy as jnp
from jax import lax
from jax.experimental import pallas as pl
from jax.experimental.pallas import tpu as pltpu
from jax.experimental.pallas import tpu_sc as plsc

ED = 64
M = 1000
ROWS = M + 1
BATCH = 1024
NB = 8
NIDX = BATCH * NB * NB * NB

NC = 2
NS = 16
NW = NC * NS
NI = NIDX // NW
ISZ = 128
NSTREAM = 4
CHUNK = ISZ * NSTREAM
NCHUNK = NI // CHUNK


def _body(table_hbm, gi_hbm, out_hbm, idx_v, rows_v, sem):
    c = lax.axis_index("c")
    s = lax.axis_index("s")
    wid = s * NC + c
    base = wid * NI

    def chunk_step(i, carry):
        off = base + i * CHUNK
        row = wid * (NI // ISZ) + i * NSTREAM
        pltpu.sync_copy(gi_hbm.at[pl.ds(row, NSTREAM)], idx_v)
        for j in range(NSTREAM):
            pltpu.async_copy(
                table_hbm.at[idx_v.at[j]],
                rows_v.at[pl.ds(j * ISZ, ISZ)],
                sem,
            )
        for j in range(NSTREAM):
            pltpu.make_async_copy(
                table_hbm.at[idx_v.at[j]],
                rows_v.at[pl.ds(j * ISZ, ISZ)],
                sem,
            ).wait()
        pltpu.sync_copy(rows_v, out_hbm.at[pl.ds(off, CHUNK)])
        return carry

    lax.fori_loop(0, NCHUNK, chunk_step, 0)


@jax.jit
def _gather(table, gi_flat):
    mesh = plsc.VectorSubcoreMesh(core_axis_name="c", subcore_axis_name="s")
    f = functools.partial(
        pl.kernel,
        mesh=mesh,
        out_type=jax.ShapeDtypeStruct((NIDX, ED), jnp.float32),
        scratch_types=[
            pltpu.VMEM((NSTREAM, ISZ), jnp.int32),
            pltpu.VMEM((CHUNK, ED), jnp.float32),
            pltpu.SemaphoreType.DMA,
        ],
        compiler_params=pltpu.CompilerParams(use_tc_tiling_on_sc=False),
    )(_body)
    return f(table, gi_flat)


def kernel(x, gi, ee):
    table = jnp.concatenate([ee, x], axis=0)
    gi_flat = gi.reshape(NIDX // ISZ, ISZ).astype(jnp.int32)
    out = _gather(table, gi_flat)
    return out.reshape(BATCH, NB, NB, NB, ED)
